# baseline (speedup 1.0000x reference)
import functools

import jax
import jax.numpy as jnp
from jax import lax
from jax.experimental import pallas as pl
from jax.experimental.pallas import tpu as pltpu
from jax.experimental.pallas import tpu_sc as plsc

B, N, K = 8, 16384, 4096
C_IN, C_MID = 128, 128
NUM_CLS = 3
BN_EPS = 1e-5

BLK = 2048
NB = N // BLK

SC_CORES, SC_SUBCORES, L = 2, 16, 16
NVREG = N // L


def _tc_body(cls_ref, f_ref, pts_ref, w1_ref, b1_ref, g_ref, be_ref, w2_ref,
             mol_ref, keys_ref, p0_ref, p1_ref, p2_ref, o0_ref, o1_ref,
             o2_ref, x0_ref, x1_ref, x2_ref):
  clst = cls_ref[0].T
  s = jnp.max(clst, axis=0)
  s = jnp.where(s == 0.0, 0.0, s)
  u = lax.bitcast_convert_type(s, jnp.int32)
  keys_ref[...] = jnp.where(s < 0.0, u, jnp.int32(0x7FFFFFFF) - u)

  inv = 1.0 / (1.0 + BN_EPS) ** 0.5
  scale = g_ref[...] * inv
  w1e = w1_ref[...] * scale
  bias = b1_ref[...] * scale + be_ref[...]
  f = f_ref[0]
  h = lax.dot_general(w1e, f, (((1,), (0,)), ((), ())),
                      preferred_element_type=jnp.float32) + bias
  h = jnp.maximum(h, 0.0)
  off = lax.dot_general(w2_ref[...], h, (((1,), (0,)), ((), ())),
                        preferred_element_type=jnp.float32)
  mol = mol_ref[...]
  lim = jnp.where(off > mol, mol, off)
  lim = jnp.where(lim < -mol, -mol, lim)
  ptst = pts_ref[0].T
  predb = ptst + lim
  for c, r in enumerate((p0_ref, p1_ref, p2_ref)):
    r[...] = predb[c]
  for c, r in enumerate((o0_ref, o1_ref, o2_ref)):
    r[...] = off[c]
  for c, r in enumerate((x0_ref, x1_ref, x2_ref)):
    r[...] = ptst[c]


def _tc_stage(cls_preds, features, points, w1, b1_c, g_c, be_c, w2, mol_c):
  grid = (B, NB)
  full = lambda b, n: (0, 0)
  row = pl.BlockSpec((BLK,), lambda b, n: (b * NB + n,))
  frow = jax.ShapeDtypeStruct((B * N,), jnp.float32)
  return pl.pallas_call(
      _tc_body,
      grid=grid,
      in_specs=[
          pl.BlockSpec((1, BLK, NUM_CLS), lambda b, n: (b, n, 0)),
          pl.BlockSpec((1, C_IN, BLK), lambda b, n: (b, 0, n)),
          pl.BlockSpec((1, BLK, 3), lambda b, n: (b, n, 0)),
          pl.BlockSpec((C_MID, C_IN), full),
          pl.BlockSpec((C_MID, 1), full),
          pl.BlockSpec((C_MID, 1), full),
          pl.BlockSpec((C_MID, 1), full),
          pl.BlockSpec((3, C_MID), full),
          pl.BlockSpec((3, 1), full),
      ],
      out_specs=[row] * 10,
      out_shape=[jax.ShapeDtypeStruct((B * N,), jnp.int32)] + [frow] * 9,
  )(cls_preds, features, points, w1, b1_c, g_c, be_c, w2, mol_c)


def _sc_body(keys_hbm, p0, p1, p2, o0, o1, o2, x0, x1, x2,
             preds_o, orig_o, offs_o,
             keys0, keys1, idx0, idx1, hist, row, outb):
  wid = lax.axis_index("s") * SC_CORES + lax.axis_index("c")

  @pl.when(wid < B)
  def _():
    b = wid
    pltpu.sync_copy(keys_hbm.at[pl.ds(b * N, N)], keys0)

    def init_iota(j, _):
      idx0[pl.ds(j * L, L)] = lax.iota(jnp.int32, L) + j * L
      return 0
    lax.fori_loop(0, NVREG, init_iota, 0)

    for p in range(4):
      src_k, src_i = (keys0, idx0) if p % 2 == 0 else (keys1, idx1)
      dst_k, dst_i = (keys1, idx1) if p % 2 == 0 else (keys0, idx0)
      shift = 8 * p

      def zero_hist(i, _):
        hist[pl.ds(i * L, L)] = jnp.zeros((L,), jnp.int32)
        return 0
      lax.fori_loop(0, 256 // L, zero_hist, 0)

      def count(j, _, src_k=src_k, shift=shift):
        k = src_k[pl.ds(j * L, L)]
        d = lax.shift_right_logical(k, shift) & 255
        cnt, lastm = plsc.scan_count(d)
        plsc.addupdate_scatter(hist, [d], cnt, mask=lastm)
        return 0
      lax.fori_loop(0, NVREG, count, 0)

      def excl_scan(i, carry):
        chunk = hist[pl.ds(i * L, L)]
        incl = plsc.cumsum(chunk)
        hist[pl.ds(i * L, L)] = incl - chunk + carry
        return carry + jnp.max(incl)
      lax.fori_loop(0, 256 // L, excl_scan, jnp.int32(0))

      def permute(j, _, src_k=src_k, src_i=src_i, dst_k=dst_k, dst_i=dst_i,
                  shift=shift):
        k = src_k[pl.ds(j * L, L)]
        iv = src_i[pl.ds(j * L, L)]
        d = lax.shift_right_logical(k, shift) & 255
        cnt, lastm = plsc.scan_count(d)
        base = plsc.load_gather(hist, [d])
        pos = base + cnt - 1
        plsc.store_scatter(dst_k, [pos], k)
        plsc.store_scatter(dst_i, [pos], iv)
        plsc.addupdate_scatter(hist, [d], cnt, mask=lastm)
        return 0
      lax.fori_loop(0, NVREG, permute, 0)

    for srcs, out_hbm in (((p0, p1, p2), preds_o), ((x0, x1, x2), orig_o),
                          ((o0, o1, o2), offs_o)):
      for c in range(3):
        pltpu.sync_copy(srcs[c].at[pl.ds(b * N, N)], row)

        def gather(j, _, c=c):
          iv = idx0[pl.ds(j * L, L)]
          v = plsc.load_gather(row, [iv])
          plsc.store_scatter(outb, [(lax.iota(jnp.int32, L) + j * L) * 3 + c],
                             v)
          return 0
        lax.fori_loop(0, K // L, gather, 0)
      pltpu.sync_copy(outb, out_hbm.at[pl.ds(b * K * 3, K * 3)])


def _sc_stage(keys, rows9):
  mesh = plsc.VectorSubcoreMesh(core_axis_name="c", subcore_axis_name="s",
                                num_cores=SC_CORES, num_subcores=SC_SUBCORES)
  outk3 = jax.ShapeDtypeStruct((B * K * 3,), jnp.float32)
  fn = pl.kernel(
      _sc_body,
      out_type=(outk3, outk3, outk3),
      mesh=mesh,
      compiler_params=pltpu.CompilerParams(needs_layout_passes=False),
      scratch_types=[
          pltpu.VMEM((N,), jnp.int32),
          pltpu.VMEM((N,), jnp.int32),
          pltpu.VMEM((N,), jnp.int32),
          pltpu.VMEM((N,), jnp.int32),
          pltpu.VMEM((256,), jnp.int32),
          pltpu.VMEM((N,), jnp.float32),
          pltpu.VMEM((K * 3,), jnp.float32),
      ],
  )
  p, o, f = fn(keys, *rows9)
  return (p.reshape(B, K, 3), o.reshape(B, K, 3), f.reshape(B, K, 3))


def kernel(points, features, cls_preds, W1, b1, gamma, beta, W2,
           max_offset_limit):
  outs = _tc_stage(
      cls_preds, features, points, W1,
      b1.reshape(C_MID, 1), gamma.reshape(C_MID, 1), beta.reshape(C_MID, 1),
      W2, max_offset_limit.reshape(3, 1))
  keys, rows9 = outs[0], outs[1:]
  return _sc_stage(keys, rows9)


# speedup vs baseline: 1.6306x; 1.6306x over previous
import functools

import jax
import jax.numpy as jnp
from jax import lax
from jax.experimental import pallas as pl
from jax.experimental.pallas import tpu as pltpu
from jax.experimental.pallas import tpu_sc as plsc

B, N, K = 8, 16384, 4096
C_IN, C_MID = 128, 128
NUM_CLS = 3
BN_EPS = 1e-5

BLK = 2048
NB = N // BLK

SC_CORES, SC_SUBCORES, L = 2, 16, 16
NVREG = N // L


def _tc_body(cls_ref, f_ref, pts_ref, w1_ref, b1_ref, g_ref, be_ref, w2_ref,
             mol_ref, keys_ref, p0_ref, p1_ref, p2_ref, o0_ref, o1_ref,
             o2_ref, x0_ref, x1_ref, x2_ref):
  clst = cls_ref[...]
  s = jnp.max(clst, axis=0)
  s = jnp.where(s == 0.0, 0.0, s)
  u = lax.bitcast_convert_type(s, jnp.int32)
  keys_ref[...] = jnp.where(s < 0.0, u, jnp.int32(0x7FFFFFFF) - u)

  inv = 1.0 / (1.0 + BN_EPS) ** 0.5
  scale = g_ref[...] * inv
  w1e = w1_ref[...] * scale
  bias = b1_ref[...] * scale + be_ref[...]
  f = f_ref[0]
  h = lax.dot_general(w1e, f, (((1,), (0,)), ((), ())),
                      preferred_element_type=jnp.float32) + bias
  h = jnp.maximum(h, 0.0)
  off = lax.dot_general(w2_ref[...], h, (((1,), (0,)), ((), ())),
                        preferred_element_type=jnp.float32)
  mol = mol_ref[...]
  lim = jnp.where(off > mol, mol, off)
  lim = jnp.where(lim < -mol, -mol, lim)
  ptst = pts_ref[...]
  predb = ptst + lim
  for c, r in enumerate((p0_ref, p1_ref, p2_ref)):
    r[...] = predb[c]
  for c, r in enumerate((o0_ref, o1_ref, o2_ref)):
    r[...] = off[c]
  for c, r in enumerate((x0_ref, x1_ref, x2_ref)):
    r[...] = ptst[c]


def _tc_stage(cls2, features, pts2, w1, b1_c, g_c, be_c, w2, mol_c):
  grid = (B, NB)
  full = lambda b, n: (0, 0)
  row = pl.BlockSpec((BLK,), lambda b, n: (b * NB + n,))
  frow = jax.ShapeDtypeStruct((B * N,), jnp.float32)
  return pl.pallas_call(
      _tc_body,
      grid=grid,
      in_specs=[
          pl.BlockSpec((NUM_CLS, BLK), lambda b, n: (0, b * NB + n)),
          pl.BlockSpec((1, C_IN, BLK), lambda b, n: (b, 0, n)),
          pl.BlockSpec((3, BLK), lambda b, n: (0, b * NB + n)),
          pl.BlockSpec((C_MID, C_IN), full),
          pl.BlockSpec((C_MID, 1), full),
          pl.BlockSpec((C_MID, 1), full),
          pl.BlockSpec((C_MID, 1), full),
          pl.BlockSpec((3, C_MID), full),
          pl.BlockSpec((3, 1), full),
      ],
      out_specs=[row] * 10,
      out_shape=[jax.ShapeDtypeStruct((B * N,), jnp.int32)] + [frow] * 9,
  )(cls2, features, pts2, w1, b1_c, g_c, be_c, w2, mol_c)


def _sc_body(keys_hbm, p0, p1, p2, o0, o1, o2, x0, x1, x2,
             preds_o, orig_o, offs_o,
             keys0, keys1, idx0, idx1, hist, row, outb):
  wid = lax.axis_index("s") * SC_CORES + lax.axis_index("c")

  @pl.when(wid < B)
  def _():
    b = wid
    pltpu.sync_copy(keys_hbm.at[pl.ds(b * N, N)], keys0)

    def init_iota(j, _):
      idx0[pl.ds(j * L, L)] = lax.iota(jnp.int32, L) + j * L
      return 0
    lax.fori_loop(0, NVREG, init_iota, 0)

    for p in range(4):
      src_k, src_i = (keys0, idx0) if p % 2 == 0 else (keys1, idx1)
      dst_k, dst_i = (keys1, idx1) if p % 2 == 0 else (keys0, idx0)
      shift = 8 * p

      def zero_hist(i, _):
        hist[pl.ds(i * L, L)] = jnp.zeros((L,), jnp.int32)
        return 0
      lax.fori_loop(0, 256 // L, zero_hist, 0)

      def count(j, _, src_k=src_k, shift=shift):
        k = src_k[pl.ds(j * L, L)]
        d = lax.shift_right_logical(k, shift) & 255
        cnt, lastm = plsc.scan_count(d)
        plsc.addupdate_scatter(hist, [d], cnt, mask=lastm)
        return 0
      lax.fori_loop(0, NVREG, count, 0)

      def excl_scan(i, carry):
        chunk = hist[pl.ds(i * L, L)]
        incl = plsc.cumsum(chunk)
        hist[pl.ds(i * L, L)] = incl - chunk + carry
        return carry + jnp.max(incl)
      lax.fori_loop(0, 256 // L, excl_scan, jnp.int32(0))

      def permute(j, _, src_k=src_k, src_i=src_i, dst_k=dst_k, dst_i=dst_i,
                  shift=shift):
        k = src_k[pl.ds(j * L, L)]
        iv = src_i[pl.ds(j * L, L)]
        d = lax.shift_right_logical(k, shift) & 255
        cnt, lastm = plsc.scan_count(d)
        base = plsc.load_gather(hist, [d])
        pos = base + cnt - 1
        plsc.store_scatter(dst_k, [pos], k)
        plsc.store_scatter(dst_i, [pos], iv)
        plsc.addupdate_scatter(hist, [d], cnt, mask=lastm)
        return 0
      lax.fori_loop(0, NVREG, permute, 0)

    for srcs, out_hbm in (((p0, p1, p2), preds_o), ((x0, x1, x2), orig_o),
                          ((o0, o1, o2), offs_o)):
      for c in range(3):
        pltpu.sync_copy(srcs[c].at[pl.ds(b * N, N)], row)

        def gather(j, _):
          iv = idx0[pl.ds(j * L, L)]
          outb[pl.ds(j * L, L)] = plsc.load_gather(row, [iv])
          return 0
        lax.fori_loop(0, K // L, gather, 0)
        pltpu.sync_copy(outb, out_hbm.at[pl.ds((c * B + b) * K, K)])


def _sc_stage(keys, rows9):
  mesh = plsc.VectorSubcoreMesh(core_axis_name="c", subcore_axis_name="s",
                                num_cores=SC_CORES, num_subcores=SC_SUBCORES)
  outk3 = jax.ShapeDtypeStruct((B * K * 3,), jnp.float32)
  fn = pl.kernel(
      _sc_body,
      out_type=(outk3, outk3, outk3),
      mesh=mesh,
      compiler_params=pltpu.CompilerParams(needs_layout_passes=False),
      scratch_types=[
          pltpu.VMEM((N,), jnp.int32),
          pltpu.VMEM((N,), jnp.int32),
          pltpu.VMEM((N,), jnp.int32),
          pltpu.VMEM((N,), jnp.int32),
          pltpu.VMEM((256,), jnp.int32),
          pltpu.VMEM((N,), jnp.float32),
          pltpu.VMEM((K,), jnp.float32),
      ],
  )
  p, o, f = fn(keys, *rows9)
  tok3 = lambda r: jnp.transpose(r.reshape(3, B, K), (1, 2, 0))
  return (tok3(p), tok3(o), tok3(f))


def kernel(points, features, cls_preds, W1, b1, gamma, beta, W2,
           max_offset_limit):
  cls2 = jnp.transpose(cls_preds, (2, 0, 1)).reshape(NUM_CLS, B * N)
  pts2 = jnp.transpose(points, (2, 0, 1)).reshape(3, B * N)
  outs = _tc_stage(
      cls2, features, pts2, W1,
      b1.reshape(C_MID, 1), gamma.reshape(C_MID, 1), beta.reshape(C_MID, 1),
      W2, max_offset_limit.reshape(3, 1))
  keys, rows9 = outs[0], outs[1:]
  return _sc_stage(keys, rows9)


# speedup vs baseline: 2.1754x; 1.3342x over previous
import functools

import jax
import jax.numpy as jnp
from jax import lax
from jax.experimental import pallas as pl
from jax.experimental.pallas import tpu as pltpu
from jax.experimental.pallas import tpu_sc as plsc

B, N, K = 8, 16384, 4096
C_IN, C_MID = 128, 128
NUM_CLS = 3
BN_EPS = 1e-5

BLK = 2048
NB = N // BLK

SC_CORES, SC_SUBCORES, L = 2, 16, 16
NVREG = N // L
BIG = 0x7FFFFFFF


def _tc_body(cls_ref, f_ref, pts_ref, w1_ref, b1_ref, g_ref, be_ref, w2_ref,
             mol_ref, keys_ref, p0_ref, p1_ref, p2_ref, o0_ref, o1_ref,
             o2_ref, x0_ref, x1_ref, x2_ref):
  clst = cls_ref[...]
  s = jnp.max(clst, axis=0)
  s = jnp.where(s == 0.0, 0.0, s)
  u = lax.bitcast_convert_type(s, jnp.int32)
  keys_ref[...] = jnp.where(s < 0.0, u, jnp.int32(0x7FFFFFFF) - u)

  inv = 1.0 / (1.0 + BN_EPS) ** 0.5
  scale = g_ref[...] * inv
  w1e = w1_ref[...] * scale
  bias = b1_ref[...] * scale + be_ref[...]
  f = f_ref[0]
  h = lax.dot_general(w1e, f, (((1,), (0,)), ((), ())),
                      preferred_element_type=jnp.float32) + bias
  h = jnp.maximum(h, 0.0)
  off = lax.dot_general(w2_ref[...], h, (((1,), (0,)), ((), ())),
                        preferred_element_type=jnp.float32)
  mol = mol_ref[...]
  lim = jnp.where(off > mol, mol, off)
  lim = jnp.where(lim < -mol, -mol, lim)
  ptst = pts_ref[...]
  predb = ptst + lim
  for c, r in enumerate((p0_ref, p1_ref, p2_ref)):
    r[...] = predb[c]
  for c, r in enumerate((o0_ref, o1_ref, o2_ref)):
    r[...] = off[c]
  for c, r in enumerate((x0_ref, x1_ref, x2_ref)):
    r[...] = ptst[c]


def _tc_stage(cls2, features, pts2, w1, b1_c, g_c, be_c, w2, mol_c):
  grid = (B, NB)
  full = lambda b, n: (0, 0)
  row = pl.BlockSpec((BLK,), lambda b, n: (b * NB + n,))
  frow = jax.ShapeDtypeStruct((B * N,), jnp.float32)
  return pl.pallas_call(
      _tc_body,
      grid=grid,
      in_specs=[
          pl.BlockSpec((NUM_CLS, BLK), lambda b, n: (0, b * NB + n)),
          pl.BlockSpec((1, C_IN, BLK), lambda b, n: (b, 0, n)),
          pl.BlockSpec((3, BLK), lambda b, n: (0, b * NB + n)),
          pl.BlockSpec((C_MID, C_IN), full),
          pl.BlockSpec((C_MID, 1), full),
          pl.BlockSpec((C_MID, 1), full),
          pl.BlockSpec((C_MID, 1), full),
          pl.BlockSpec((3, C_MID), full),
          pl.BlockSpec((3, 1), full),
      ],
      out_specs=[row] * 10,
      out_shape=[jax.ShapeDtypeStruct((B * N,), jnp.int32)] + [frow] * 9,
  )(cls2, features, pts2, w1, b1_c, g_c, be_c, w2, mol_c)


def _sc_body(keys_hbm, p0, p1, p2, o0, o1, o2, x0, x1, x2,
             preds_o, orig_o, offs_o,
             keys0, keys1, idx0, idx1, hist, row, outb):
  wid = lax.axis_index("s") * SC_CORES + lax.axis_index("c")

  def zero_hist(nbins):
    def body(i, _):
      hist[pl.ds(i * L, L)] = jnp.zeros((L,), jnp.int32)
      return 0
    lax.fori_loop(0, nbins // L, body, 0)

  def count_sweep(src_k, shift, dmask, n):
    def body(j, _):
      k = src_k[pl.ds(j * L, L)]
      valid = (j * L + lax.iota(jnp.int32, L)) < n
      d = lax.shift_right_logical(k, shift) & dmask
      cnt, lastm = plsc.scan_count(d, mask=valid)
      plsc.addupdate_scatter(hist, [d], cnt, mask=lastm)
      return 0
    lax.fori_loop(0, (n + L - 1) // L, body, 0)

  def excl_scan(nbins):
    def body(i, carry):
      run, m = carry
      chunk = hist[pl.ds(i * L, L)]
      incl = plsc.cumsum(chunk) + run
      hist[pl.ds(i * L, L)] = incl - chunk
      m = jnp.minimum(m, jnp.min(jnp.where(incl >= K, incl, BIG)))
      return jnp.max(incl), m
    _, m = lax.fori_loop(0, nbins // L, body,
                         (jnp.int32(0), jnp.int32(BIG)))
    return m

  def permute_sweep(src_k, src_i, dst_k, dst_i, shift, dmask, n,
                    store_keys=True):
    def body(j, _):
      k = src_k[pl.ds(j * L, L)]
      iv = src_i[pl.ds(j * L, L)]
      valid = (j * L + lax.iota(jnp.int32, L)) < n
      d = lax.shift_right_logical(k, shift) & dmask
      cnt, lastm = plsc.scan_count(d, mask=valid)
      base = plsc.load_gather(hist, [d])
      pos = base + cnt - 1
      if store_keys:
        plsc.store_scatter(dst_k, [pos], k, mask=valid)
      plsc.store_scatter(dst_i, [pos], iv, mask=valid)
      plsc.addupdate_scatter(hist, [d], cnt, mask=lastm)
      return 0
    lax.fori_loop(0, (n + L - 1) // L, body, 0)

  @pl.when(wid < B)
  def _():
    b = wid
    pltpu.sync_copy(keys_hbm.at[pl.ds(b * N, N)], keys0)

    def init_iota(j, _):
      idx0[pl.ds(j * L, L)] = lax.iota(jnp.int32, L) + j * L
      return 0
    lax.fori_loop(0, NVREG, init_iota, 0)

    zero_hist(2048)
    count_sweep(keys0, 21, 2047, N)
    m = excl_scan(2048)
    permute_sweep(keys0, idx0, keys1, idx1, 21, 2047, N)

    zero_hist(2048)
    count_sweep(keys1, 0, 2047, m)
    excl_scan(2048)
    permute_sweep(keys1, idx1, keys0, idx0, 0, 2047, m)

    zero_hist(1024)
    count_sweep(keys0, 11, 1023, m)
    excl_scan(1024)
    permute_sweep(keys0, idx0, keys1, idx1, 11, 1023, m)

    zero_hist(2048)
    count_sweep(keys1, 21, 2047, m)
    excl_scan(2048)
    permute_sweep(keys1, idx1, keys0, idx0, 21, 2047, m, store_keys=False)

    for srcs, out_hbm in (((p0, p1, p2), preds_o), ((x0, x1, x2), orig_o),
                          ((o0, o1, o2), offs_o)):
      for c in range(3):
        pltpu.sync_copy(srcs[c].at[pl.ds(b * N, N)], row)

        def gather(j, _):
          iv = idx0[pl.ds(j * L, L)]
          outb[pl.ds(j * L, L)] = plsc.load_gather(row, [iv])
          return 0
        lax.fori_loop(0, K // L, gather, 0)
        pltpu.sync_copy(outb, out_hbm.at[pl.ds((c * B + b) * K, K)])


def _sc_stage(keys, rows9):
  mesh = plsc.VectorSubcoreMesh(core_axis_name="c", subcore_axis_name="s",
                                num_cores=SC_CORES, num_subcores=SC_SUBCORES)
  outk3 = jax.ShapeDtypeStruct((B * K * 3,), jnp.float32)
  fn = pl.kernel(
      _sc_body,
      out_type=(outk3, outk3, outk3),
      mesh=mesh,
      compiler_params=pltpu.CompilerParams(needs_layout_passes=False),
      scratch_types=[
          pltpu.VMEM((N,), jnp.int32),
          pltpu.VMEM((N,), jnp.int32),
          pltpu.VMEM((N,), jnp.int32),
          pltpu.VMEM((N,), jnp.int32),
          pltpu.VMEM((2048,), jnp.int32),
          pltpu.VMEM((N,), jnp.float32),
          pltpu.VMEM((K,), jnp.float32),
      ],
  )
  p, o, f = fn(keys, *rows9)
  tok3 = lambda r: jnp.transpose(r.reshape(3, B, K), (1, 2, 0))
  return (tok3(p), tok3(o), tok3(f))


def kernel(points, features, cls_preds, W1, b1, gamma, beta, W2,
           max_offset_limit):
  cls2 = jnp.transpose(cls_preds, (2, 0, 1)).reshape(NUM_CLS, B * N)
  pts2 = jnp.transpose(points, (2, 0, 1)).reshape(3, B * N)
  outs = _tc_stage(
      cls2, features, pts2, W1,
      b1.reshape(C_MID, 1), gamma.reshape(C_MID, 1), beta.reshape(C_MID, 1),
      W2, max_offset_limit.reshape(3, 1))
  keys, rows9 = outs[0], outs[1:]
  return _sc_stage(keys, rows9)


# speedup vs baseline: 2.8183x; 1.2955x over previous
import functools

import jax
import jax.numpy as jnp
from jax import lax
from jax.experimental import pallas as pl
from jax.experimental.pallas import tpu as pltpu
from jax.experimental.pallas import tpu_sc as plsc

B, N, K = 8, 16384, 4096
C_IN, C_MID = 128, 128
NUM_CLS = 3
BN_EPS = 1e-5

BLK = 2048
NB = N // BLK

SC_CORES, SC_SUBCORES, L = 2, 16, 16
NVREG = N // L
BIG = 0x7FFFFFFF


def _tc_body(cls_ref, f_ref, pts_ref, w1_ref, b1_ref, g_ref, be_ref, w2_ref,
             mol_ref, keys_ref, p0_ref, p1_ref, p2_ref, o0_ref, o1_ref,
             o2_ref, x0_ref, x1_ref, x2_ref):
  clst = cls_ref[...]
  s = jnp.max(clst, axis=0)
  s = jnp.where(s == 0.0, 0.0, s)
  u = lax.bitcast_convert_type(s, jnp.int32)
  keys_ref[...] = jnp.where(s < 0.0, u, jnp.int32(0x7FFFFFFF) - u)

  inv = 1.0 / (1.0 + BN_EPS) ** 0.5
  scale = g_ref[...] * inv
  w1e = w1_ref[...] * scale
  bias = b1_ref[...] * scale + be_ref[...]
  f = f_ref[0]
  h = lax.dot_general(w1e, f, (((1,), (0,)), ((), ())),
                      preferred_element_type=jnp.float32) + bias
  h = jnp.maximum(h, 0.0)
  off = lax.dot_general(w2_ref[...], h, (((1,), (0,)), ((), ())),
                        preferred_element_type=jnp.float32)
  mol = mol_ref[...]
  lim = jnp.where(off > mol, mol, off)
  lim = jnp.where(lim < -mol, -mol, lim)
  ptst = pts_ref[...]
  predb = ptst + lim
  for c, r in enumerate((p0_ref, p1_ref, p2_ref)):
    r[...] = predb[c]
  for c, r in enumerate((o0_ref, o1_ref, o2_ref)):
    r[...] = off[c]
  for c, r in enumerate((x0_ref, x1_ref, x2_ref)):
    r[...] = ptst[c]


def _tc_stage(cls2, features, pts2, w1, b1_c, g_c, be_c, w2, mol_c):
  grid = (B, NB)
  full = lambda b, n: (0, 0)
  row = pl.BlockSpec((BLK,), lambda b, n: (b * NB + n,))
  frow = jax.ShapeDtypeStruct((B * N,), jnp.float32)
  return pl.pallas_call(
      _tc_body,
      grid=grid,
      in_specs=[
          pl.BlockSpec((NUM_CLS, BLK), lambda b, n: (0, b * NB + n)),
          pl.BlockSpec((1, C_IN, BLK), lambda b, n: (b, 0, n)),
          pl.BlockSpec((3, BLK), lambda b, n: (0, b * NB + n)),
          pl.BlockSpec((C_MID, C_IN), full),
          pl.BlockSpec((C_MID, 1), full),
          pl.BlockSpec((C_MID, 1), full),
          pl.BlockSpec((C_MID, 1), full),
          pl.BlockSpec((3, C_MID), full),
          pl.BlockSpec((3, 1), full),
      ],
      out_specs=[row] * 10,
      out_shape=[jax.ShapeDtypeStruct((B * N,), jnp.int32)] + [frow] * 9,
  )(cls2, features, pts2, w1, b1_c, g_c, be_c, w2, mol_c)


def _sc_body(keys_hbm, p0, p1, p2, o0, o1, o2, x0, x1, x2,
             preds_o, orig_o, offs_o,
             keys0, keys1, idx0, idx1, hist, row, outb,
             kq, ghist, ck, ci, cbuf,
             hists_sh, ckeys_sh, cidx_sh, cnts_sh, idxt_sh):
  c = lax.axis_index("c")
  s = lax.axis_index("s")
  b = c + 2 * (s % 4)
  bb = s % 4
  q = s // 4

  def zero_hist(nbins):
    def body(i, _):
      hist[pl.ds(i * L, L)] = jnp.zeros((L,), jnp.int32)
      return 0
    lax.fori_loop(0, nbins // L, body, 0)

  def count_sweep(src_k, shift, dmask, n, start=0):
    def body(j, _):
      k = src_k[pl.ds(start + j * L, L)]
      valid = (j * L + lax.iota(jnp.int32, L)) < n
      d = lax.shift_right_logical(k, shift) & dmask
      cnt, lastm = plsc.scan_count(d, mask=valid)
      plsc.addupdate_scatter(hist, [d], cnt, mask=lastm)
      return 0
    lax.fori_loop(0, (n + L - 1) // L, body, 0)

  def excl_scan(nbins):
    def body(i, carry):
      run, m = carry
      chunk = hist[pl.ds(i * L, L)]
      incl = plsc.cumsum(chunk) + run
      hist[pl.ds(i * L, L)] = incl - chunk
      m = jnp.minimum(m, jnp.min(jnp.where(incl >= K, incl, BIG)))
      return jnp.max(incl), m
    _, m = lax.fori_loop(0, nbins // L, body,
                         (jnp.int32(0), jnp.int32(BIG)))
    return m

  def permute_sweep(src_k, src_i, dst_k, dst_i, shift, dmask, n,
                    store_keys=True, start=0):
    def body(j, _):
      k = src_k[pl.ds(start + j * L, L)]
      iv = src_i[pl.ds(start + j * L, L)]
      valid = (j * L + lax.iota(jnp.int32, L)) < n
      d = lax.shift_right_logical(k, shift) & dmask
      cnt, lastm = plsc.scan_count(d, mask=valid)
      base = plsc.load_gather(hist, [d])
      pos = base + cnt - 1
      if store_keys:
        plsc.store_scatter(dst_k, [pos], k, mask=valid)
      plsc.store_scatter(dst_i, [pos], iv, mask=valid)
      plsc.addupdate_scatter(hist, [d], cnt, mask=lastm)
      return 0
    lax.fori_loop(0, (n + L - 1) // L, body, 0)

  QN = N // 4
  QV = QN // L

  pltpu.sync_copy(keys_hbm.at[pl.ds(b * N + q * QN, QN)], kq)
  zero_hist(2048)
  count_sweep(kq, 21, 2047, QN)
  pltpu.sync_copy(hist, hists_sh.at[pl.ds((bb * 4 + q) * 2048, 2048)])
  plsc.subcore_barrier()

  pltpu.sync_copy(hists_sh.at[pl.ds(bb * 4 * 2048, 4 * 2048)], ghist)

  def gscan(i, carry):
    run, db = carry
    tot = (ghist[pl.ds(i * L, L)] + ghist[pl.ds(2048 + i * L, L)] +
           ghist[pl.ds(4096 + i * L, L)] + ghist[pl.ds(6144 + i * L, L)])
    incl = plsc.cumsum(tot) + run
    lane_d = i * L + lax.iota(jnp.int32, L)
    db = jnp.minimum(db, jnp.min(jnp.where(incl >= K, lane_d, BIG)))
    return jnp.max(incl), db
  _, d_bnd = lax.fori_loop(0, 2048 // L, gscan,
                           (jnp.int32(0), jnp.int32(BIG)))

  def compact(j, base):
    k = kq[pl.ds(j * L, L)]
    iv = q * QN + j * L + lax.iota(jnp.int32, L)
    d = lax.shift_right_logical(k, 21) & 2047
    msk = d <= d_bnd
    cnt = plsc.cumsum(jnp.where(msk, 1, 0))
    pos = base + cnt - 1
    plsc.store_scatter(ck, [pos], k, mask=msk)
    plsc.store_scatter(ci, [pos], iv, mask=msk)
    return base + jnp.max(cnt)
  cntq = lax.fori_loop(0, QV, compact, jnp.int32(0))

  pltpu.sync_copy(ck, ckeys_sh.at[pl.ds((bb * 4 + q) * QN, QN)])
  pltpu.sync_copy(ci, cidx_sh.at[pl.ds((bb * 4 + q) * QN, QN)])
  cbuf[pl.ds(0, L)] = jnp.full((L,), cntq, jnp.int32)
  pltpu.sync_copy(cbuf.at[pl.ds(0, L)], cnts_sh.at[pl.ds((bb * 4 + q) * L, L)])
  plsc.subcore_barrier()

  @pl.when(q == 0)
  def _():
    pltpu.sync_copy(ckeys_sh.at[pl.ds(bb * 4 * QN, 4 * QN)], keys1)
    pltpu.sync_copy(cidx_sh.at[pl.ds(bb * 4 * QN, 4 * QN)], idx1)
    pltpu.sync_copy(cnts_sh.at[pl.ds(bb * 4 * L, 4 * L)], cbuf)
    cqs = [jnp.max(cbuf[pl.ds(qq * L, L)]) for qq in range(4)]
    m = cqs[0] + cqs[1] + cqs[2] + cqs[3]

    zero_hist(2048)
    for qq in range(4):
      count_sweep(keys1, 0, 2047, cqs[qq], start=qq * QN)
    excl_scan(2048)
    for qq in range(4):
      permute_sweep(keys1, idx1, keys0, idx0, 0, 2047, cqs[qq],
                    start=qq * QN)

    zero_hist(1024)
    count_sweep(keys0, 11, 1023, m)
    excl_scan(1024)
    permute_sweep(keys0, idx0, keys1, idx1, 11, 1023, m)

    zero_hist(2048)
    count_sweep(keys1, 21, 2047, m)
    excl_scan(2048)
    permute_sweep(keys1, idx1, keys0, idx0, 21, 2047, m, store_keys=False)

    pltpu.sync_copy(idx0.at[pl.ds(0, K)], idxt_sh.at[pl.ds(bb * K, K)])
  plsc.subcore_barrier()

  pltpu.sync_copy(idxt_sh.at[pl.ds(bb * K, K)], ck)
  rows9 = ((p0, preds_o, 0), (p1, preds_o, 1), (p2, preds_o, 2),
           (x0, orig_o, 0), (x1, orig_o, 1), (x2, orig_o, 2),
           (o0, offs_o, 0), (o1, offs_o, 1), (o2, offs_o, 2))
  assign = (1, 2, 3, 1, 2, 3, 1, 2, 3)
  for ridx in range(9):
    src, out_hbm, cc = rows9[ridx]
    @pl.when(q == assign[ridx])
    def _(src=src, out_hbm=out_hbm, cc=cc):
      pltpu.sync_copy(src.at[pl.ds(b * N, N)], row)

      def gather(j, _):
        iv = ck[pl.ds(j * L, L)]
        outb[pl.ds(j * L, L)] = plsc.load_gather(row, [iv])
        return 0
      lax.fori_loop(0, K // L, gather, 0)
      pltpu.sync_copy(outb, out_hbm.at[pl.ds((cc * B + b) * K, K)])


def _sc_stage(keys, rows9):
  mesh = plsc.VectorSubcoreMesh(core_axis_name="c", subcore_axis_name="s",
                                num_cores=SC_CORES, num_subcores=SC_SUBCORES)
  outk3 = jax.ShapeDtypeStruct((B * K * 3,), jnp.float32)
  fn = pl.kernel(
      _sc_body,
      out_type=(outk3, outk3, outk3),
      mesh=mesh,
      compiler_params=pltpu.CompilerParams(needs_layout_passes=False),
      scratch_types=[
          pltpu.VMEM((N,), jnp.int32),
          pltpu.VMEM((N,), jnp.int32),
          pltpu.VMEM((N,), jnp.int32),
          pltpu.VMEM((N,), jnp.int32),
          pltpu.VMEM((2048,), jnp.int32),
          pltpu.VMEM((N,), jnp.float32),
          pltpu.VMEM((K,), jnp.float32),
          pltpu.VMEM((N // 4,), jnp.int32),
          pltpu.VMEM((4 * 2048,), jnp.int32),
          pltpu.VMEM((N // 4,), jnp.int32),
          pltpu.VMEM((N // 4,), jnp.int32),
          pltpu.VMEM((64,), jnp.int32),
          pltpu.VMEM_SHARED((4 * 4 * 2048,), jnp.int32),
          pltpu.VMEM_SHARED((4 * N,), jnp.int32),
          pltpu.VMEM_SHARED((4 * N,), jnp.int32),
          pltpu.VMEM_SHARED((4 * 4 * 16,), jnp.int32),
          pltpu.VMEM_SHARED((4 * K,), jnp.int32),
      ],
  )
  p, o, f = fn(keys, *rows9)
  tok3 = lambda r: jnp.transpose(r.reshape(3, B, K), (1, 2, 0))
  return (tok3(p), tok3(o), tok3(f))


def kernel(points, features, cls_preds, W1, b1, gamma, beta, W2,
           max_offset_limit):
  cls2 = jnp.transpose(cls_preds, (2, 0, 1)).reshape(NUM_CLS, B * N)
  pts2 = jnp.transpose(points, (2, 0, 1)).reshape(3, B * N)
  outs = _tc_stage(
      cls2, features, pts2, W1,
      b1.reshape(C_MID, 1), gamma.reshape(C_MID, 1), beta.reshape(C_MID, 1),
      W2, max_offset_limit.reshape(3, 1))
  keys, rows9 = outs[0], outs[1:]
  return _sc_stage(keys, rows9)


# speedup vs baseline: 3.2480x; 1.1524x over previous
import functools

import jax
import jax.numpy as jnp
from jax import lax
from jax.experimental import pallas as pl
from jax.experimental.pallas import tpu as pltpu
from jax.experimental.pallas import tpu_sc as plsc

B, N, K = 8, 16384, 4096
C_IN, C_MID = 128, 128
NUM_CLS = 3
BN_EPS = 1e-5

BLK = 2048
NB = N // BLK

SC_CORES, SC_SUBCORES, L = 2, 16, 16
NVREG = N // L
BIG = 0x7FFFFFFF


def _tc_keys_body(cls_ref, keys_ref):
  clst = cls_ref[...]
  s = jnp.max(clst, axis=0)
  s = jnp.where(s == 0.0, 0.0, s)
  u = lax.bitcast_convert_type(s, jnp.int32)
  keys_ref[...] = jnp.where(s < 0.0, u, jnp.int32(0x7FFFFFFF) - u)


def _tc_keys_stage(cls2):
  return pl.pallas_call(
      _tc_keys_body,
      grid=(B * NB,),
      in_specs=[pl.BlockSpec((NUM_CLS, BLK), lambda i: (0, i))],
      out_specs=pl.BlockSpec((BLK,), lambda i: (i,)),
      out_shape=jax.ShapeDtypeStruct((B * N,), jnp.int32),
  )(cls2)


def _tc_body(f_ref, pts_ref, w1_ref, b1_ref, g_ref, be_ref, w2_ref,
             mol_ref, p0_ref, p1_ref, p2_ref, o0_ref, o1_ref,
             o2_ref, x0_ref, x1_ref, x2_ref):
  inv = 1.0 / (1.0 + BN_EPS) ** 0.5
  scale = g_ref[...] * inv
  w1e = w1_ref[...] * scale
  bias = b1_ref[...] * scale + be_ref[...]
  f = f_ref[0]
  h = lax.dot_general(w1e, f, (((1,), (0,)), ((), ())),
                      preferred_element_type=jnp.float32) + bias
  h = jnp.maximum(h, 0.0)
  off = lax.dot_general(w2_ref[...], h, (((1,), (0,)), ((), ())),
                        preferred_element_type=jnp.float32)
  mol = mol_ref[...]
  lim = jnp.where(off > mol, mol, off)
  lim = jnp.where(lim < -mol, -mol, lim)
  ptst = pts_ref[...]
  predb = ptst + lim
  for c, r in enumerate((p0_ref, p1_ref, p2_ref)):
    r[...] = predb[c]
  for c, r in enumerate((o0_ref, o1_ref, o2_ref)):
    r[...] = off[c]
  for c, r in enumerate((x0_ref, x1_ref, x2_ref)):
    r[...] = ptst[c]


def _tc_stage(features, pts2, w1, b1_c, g_c, be_c, w2, mol_c):
  grid = (B, NB)
  full = lambda b, n: (0, 0)
  row = pl.BlockSpec((BLK,), lambda b, n: (b * NB + n,))
  frow = jax.ShapeDtypeStruct((B * N,), jnp.float32)
  return pl.pallas_call(
      _tc_body,
      grid=grid,
      in_specs=[
          pl.BlockSpec((1, C_IN, BLK), lambda b, n: (b, 0, n)),
          pl.BlockSpec((3, BLK), lambda b, n: (0, b * NB + n)),
          pl.BlockSpec((C_MID, C_IN), full),
          pl.BlockSpec((C_MID, 1), full),
          pl.BlockSpec((C_MID, 1), full),
          pl.BlockSpec((C_MID, 1), full),
          pl.BlockSpec((3, C_MID), full),
          pl.BlockSpec((3, 1), full),
      ],
      out_specs=[row] * 9,
      out_shape=[frow] * 9,
  )(features, pts2, w1, b1_c, g_c, be_c, w2, mol_c)


def _sc_sort_body(keys_hbm, idx_o,
                  keys0, keys1, idx0, idx1, hist,
                  kq, ghist, ck, ci, cbuf,
                  hists_sh, ckeys_sh, cidx_sh, cnts_sh):
  c = lax.axis_index("c")
  s = lax.axis_index("s")
  b = c + 2 * (s % 4)
  bb = s % 4
  q = s // 4

  def zero_hist(nbins):
    def body(i, _):
      hist[pl.ds(i * L, L)] = jnp.zeros((L,), jnp.int32)
      return 0
    lax.fori_loop(0, nbins // L, body, 0)

  def count_sweep(src_k, shift, dmask, n, start=0):
    def body(j, _):
      k = src_k[pl.ds(start + j * L, L)]
      valid = (j * L + lax.iota(jnp.int32, L)) < n
      d = lax.shift_right_logical(k, shift) & dmask
      cnt, lastm = plsc.scan_count(d, mask=valid)
      plsc.addupdate_scatter(hist, [d], cnt, mask=lastm)
      return 0
    lax.fori_loop(0, (n + L - 1) // L, body, 0)

  def excl_scan(nbins):
    def body(i, carry):
      run, m = carry
      chunk = hist[pl.ds(i * L, L)]
      incl = plsc.cumsum(chunk) + run
      hist[pl.ds(i * L, L)] = incl - chunk
      m = jnp.minimum(m, jnp.min(jnp.where(incl >= K, incl, BIG)))
      return jnp.max(incl), m
    _, m = lax.fori_loop(0, nbins // L, body,
                         (jnp.int32(0), jnp.int32(BIG)))
    return m

  def permute_sweep(src_k, src_i, dst_k, dst_i, shift, dmask, n,
                    store_keys=True, start=0):
    def body(j, _):
      k = src_k[pl.ds(start + j * L, L)]
      iv = src_i[pl.ds(start + j * L, L)]
      valid = (j * L + lax.iota(jnp.int32, L)) < n
      d = lax.shift_right_logical(k, shift) & dmask
      cnt, lastm = plsc.scan_count(d, mask=valid)
      base = plsc.load_gather(hist, [d])
      pos = base + cnt - 1
      if store_keys:
        plsc.store_scatter(dst_k, [pos], k, mask=valid)
      plsc.store_scatter(dst_i, [pos], iv, mask=valid)
      plsc.addupdate_scatter(hist, [d], cnt, mask=lastm)
      return 0
    lax.fori_loop(0, (n + L - 1) // L, body, 0)

  QN = N // 4
  QV = QN // L

  pltpu.sync_copy(keys_hbm.at[pl.ds(b * N + q * QN, QN)], kq)
  zero_hist(2048)
  count_sweep(kq, 21, 2047, QN)
  pltpu.sync_copy(hist, hists_sh.at[pl.ds((bb * 4 + q) * 2048, 2048)])
  plsc.subcore_barrier()

  pltpu.sync_copy(hists_sh.at[pl.ds(bb * 4 * 2048, 4 * 2048)], ghist)

  def gscan(i, carry):
    run, db = carry
    tot = (ghist[pl.ds(i * L, L)] + ghist[pl.ds(2048 + i * L, L)] +
           ghist[pl.ds(4096 + i * L, L)] + ghist[pl.ds(6144 + i * L, L)])
    incl = plsc.cumsum(tot) + run
    lane_d = i * L + lax.iota(jnp.int32, L)
    db = jnp.minimum(db, jnp.min(jnp.where(incl >= K, lane_d, BIG)))
    return jnp.max(incl), db
  _, d_bnd = lax.fori_loop(0, 2048 // L, gscan,
                           (jnp.int32(0), jnp.int32(BIG)))

  def compact(j, base):
    k = kq[pl.ds(j * L, L)]
    iv = q * QN + j * L + lax.iota(jnp.int32, L)
    d = lax.shift_right_logical(k, 21) & 2047
    msk = d <= d_bnd
    cnt = plsc.cumsum(jnp.where(msk, 1, 0))
    pos = base + cnt - 1
    plsc.store_scatter(ck, [pos], k, mask=msk)
    plsc.store_scatter(ci, [pos], iv, mask=msk)
    return base + jnp.max(cnt)
  cntq = lax.fori_loop(0, QV, compact, jnp.int32(0))

  pltpu.sync_copy(ck, ckeys_sh.at[pl.ds((bb * 4 + q) * QN, QN)])
  pltpu.sync_copy(ci, cidx_sh.at[pl.ds((bb * 4 + q) * QN, QN)])
  cbuf[pl.ds(0, L)] = jnp.full((L,), cntq, jnp.int32)
  pltpu.sync_copy(cbuf.at[pl.ds(0, L)], cnts_sh.at[pl.ds((bb * 4 + q) * L, L)])
  plsc.subcore_barrier()

  @pl.when(q == 0)
  def _():
    pltpu.sync_copy(ckeys_sh.at[pl.ds(bb * 4 * QN, 4 * QN)], keys1)
    pltpu.sync_copy(cidx_sh.at[pl.ds(bb * 4 * QN, 4 * QN)], idx1)
    pltpu.sync_copy(cnts_sh.at[pl.ds(bb * 4 * L, 4 * L)], cbuf)
    cqs = [jnp.max(cbuf[pl.ds(qq * L, L)]) for qq in range(4)]
    m = cqs[0] + cqs[1] + cqs[2] + cqs[3]

    zero_hist(2048)
    for qq in range(4):
      count_sweep(keys1, 0, 2047, cqs[qq], start=qq * QN)
    excl_scan(2048)
    for qq in range(4):
      permute_sweep(keys1, idx1, keys0, idx0, 0, 2047, cqs[qq],
                    start=qq * QN)

    zero_hist(1024)
    count_sweep(keys0, 11, 1023, m)
    excl_scan(1024)
    permute_sweep(keys0, idx0, keys1, idx1, 11, 1023, m)

    zero_hist(2048)
    count_sweep(keys1, 21, 2047, m)
    excl_scan(2048)
    permute_sweep(keys1, idx1, keys0, idx0, 21, 2047, m, store_keys=False)

    pltpu.sync_copy(idx0.at[pl.ds(0, K)], idx_o.at[pl.ds(b * K, K)])


def _sc_sort_stage(keys):
  mesh = plsc.VectorSubcoreMesh(core_axis_name="c", subcore_axis_name="s",
                                num_cores=SC_CORES, num_subcores=SC_SUBCORES)
  fn = pl.kernel(
      _sc_sort_body,
      out_type=jax.ShapeDtypeStruct((B * K,), jnp.int32),
      mesh=mesh,
      compiler_params=pltpu.CompilerParams(needs_layout_passes=False),
      scratch_types=[
          pltpu.VMEM((N,), jnp.int32),
          pltpu.VMEM((N,), jnp.int32),
          pltpu.VMEM((N,), jnp.int32),
          pltpu.VMEM((N,), jnp.int32),
          pltpu.VMEM((2048,), jnp.int32),
          pltpu.VMEM((N // 4,), jnp.int32),
          pltpu.VMEM((4 * 2048,), jnp.int32),
          pltpu.VMEM((N // 4,), jnp.int32),
          pltpu.VMEM((N // 4,), jnp.int32),
          pltpu.VMEM((64,), jnp.int32),
          pltpu.VMEM_SHARED((4 * 4 * 2048,), jnp.int32),
          pltpu.VMEM_SHARED((4 * N,), jnp.int32),
          pltpu.VMEM_SHARED((4 * N,), jnp.int32),
          pltpu.VMEM_SHARED((4 * 4 * 16,), jnp.int32),
      ],
  )
  return fn(keys)


def _sc_gather_body(idx_hbm, p0, p1, p2, o0, o1, o2, x0, x1, x2,
                    preds_o, orig_o, offs_o, ck, row, outb):
  c = lax.axis_index("c")
  s = lax.axis_index("s")
  b = c + 2 * (s % 4)
  q = s // 4

  pltpu.sync_copy(idx_hbm.at[pl.ds(b * K, K)], ck)
  rows9 = ((p0, preds_o, 0), (p1, preds_o, 1), (p2, preds_o, 2),
           (x0, orig_o, 0), (x1, orig_o, 1), (x2, orig_o, 2),
           (o0, offs_o, 0), (o1, offs_o, 1), (o2, offs_o, 2))
  assign = (0, 1, 2, 3, 0, 1, 2, 3, 0)
  for ridx in range(9):
    src, out_hbm, cc = rows9[ridx]
    @pl.when(q == assign[ridx])
    def _(src=src, out_hbm=out_hbm, cc=cc):
      pltpu.sync_copy(src.at[pl.ds(b * N, N)], row)

      def gather(j, _):
        iv = ck[pl.ds(j * L, L)]
        outb[pl.ds(j * L, L)] = plsc.load_gather(row, [iv])
        return 0
      lax.fori_loop(0, K // L, gather, 0)
      pltpu.sync_copy(outb, out_hbm.at[pl.ds((cc * B + b) * K, K)])


def _sc_gather_stage(idx, rows9):
  mesh = plsc.VectorSubcoreMesh(core_axis_name="c", subcore_axis_name="s",
                                num_cores=SC_CORES, num_subcores=SC_SUBCORES)
  outk3 = jax.ShapeDtypeStruct((B * K * 3,), jnp.float32)
  fn = pl.kernel(
      _sc_gather_body,
      out_type=(outk3, outk3, outk3),
      mesh=mesh,
      compiler_params=pltpu.CompilerParams(needs_layout_passes=False),
      scratch_types=[
          pltpu.VMEM((K,), jnp.int32),
          pltpu.VMEM((N,), jnp.float32),
          pltpu.VMEM((K,), jnp.float32),
      ],
  )
  p, o, f = fn(idx, *rows9)
  tok3 = lambda r: jnp.transpose(r.reshape(3, B, K), (1, 2, 0))
  return (tok3(p), tok3(o), tok3(f))


def kernel(points, features, cls_preds, W1, b1, gamma, beta, W2,
           max_offset_limit):
  cls2 = jnp.transpose(cls_preds, (2, 0, 1)).reshape(NUM_CLS, B * N)
  pts2 = jnp.transpose(points, (2, 0, 1)).reshape(3, B * N)
  keys = _tc_keys_stage(cls2)
  idx = _sc_sort_stage(keys)
  rows9 = _tc_stage(
      features, pts2, W1,
      b1.reshape(C_MID, 1), gamma.reshape(C_MID, 1), beta.reshape(C_MID, 1),
      W2, max_offset_limit.reshape(3, 1))
  return _sc_gather_stage(idx, rows9)


# speedup vs baseline: 3.3466x; 1.0304x over previous
import functools

import jax
import jax.numpy as jnp
from jax import lax
from jax.experimental import pallas as pl
from jax.experimental.pallas import tpu as pltpu
from jax.experimental.pallas import tpu_sc as plsc

B, N, K = 8, 16384, 4096
C_IN, C_MID = 128, 128
NUM_CLS = 3
BN_EPS = 1e-5

BLK = 2048
NB = N // BLK

SC_CORES, SC_SUBCORES, L = 2, 16, 16
NVREG = N // L
BIG = 0x7FFFFFFF


def _tc_keys_body(cls_ref, keys_ref):
  clst = cls_ref[...]
  s = jnp.max(clst, axis=0)
  s = jnp.where(s == 0.0, 0.0, s)
  u = lax.bitcast_convert_type(s, jnp.int32)
  keys_ref[...] = jnp.where(s < 0.0, u, jnp.int32(0x7FFFFFFF) - u)


def _tc_keys_stage(cls2):
  return pl.pallas_call(
      _tc_keys_body,
      grid=(B * NB,),
      in_specs=[pl.BlockSpec((NUM_CLS, BLK), lambda i: (0, i))],
      out_specs=pl.BlockSpec((BLK,), lambda i: (i,)),
      out_shape=jax.ShapeDtypeStruct((B * N,), jnp.int32),
  )(cls2)


def _tc_body(f_ref, pts_ref, w1_ref, b1_ref, g_ref, be_ref, w2_ref,
             mol_ref, p0_ref, p1_ref, p2_ref, o0_ref, o1_ref,
             o2_ref, x0_ref, x1_ref, x2_ref):
  inv = 1.0 / (1.0 + BN_EPS) ** 0.5
  scale = g_ref[...] * inv
  w1e = w1_ref[...] * scale
  bias = b1_ref[...] * scale + be_ref[...]
  f = f_ref[0]
  h = lax.dot_general(w1e, f, (((1,), (0,)), ((), ())),
                      preferred_element_type=jnp.float32) + bias
  h = jnp.maximum(h, 0.0)
  off = lax.dot_general(w2_ref[...], h, (((1,), (0,)), ((), ())),
                        preferred_element_type=jnp.float32)
  mol = mol_ref[...]
  lim = jnp.where(off > mol, mol, off)
  lim = jnp.where(lim < -mol, -mol, lim)
  ptst = pts_ref[...]
  predb = ptst + lim
  for c, r in enumerate((p0_ref, p1_ref, p2_ref)):
    r[...] = predb[c]
  for c, r in enumerate((o0_ref, o1_ref, o2_ref)):
    r[...] = off[c]
  for c, r in enumerate((x0_ref, x1_ref, x2_ref)):
    r[...] = ptst[c]


def _tc_stage(features, pts2, w1, b1_c, g_c, be_c, w2, mol_c):
  grid = (B, NB)
  full = lambda b, n: (0, 0)
  row = pl.BlockSpec((BLK,), lambda b, n: (b * NB + n,))
  frow = jax.ShapeDtypeStruct((B * N,), jnp.float32)
  return pl.pallas_call(
      _tc_body,
      grid=grid,
      in_specs=[
          pl.BlockSpec((1, C_IN, BLK), lambda b, n: (b, 0, n)),
          pl.BlockSpec((3, BLK), lambda b, n: (0, b * NB + n)),
          pl.BlockSpec((C_MID, C_IN), full),
          pl.BlockSpec((C_MID, 1), full),
          pl.BlockSpec((C_MID, 1), full),
          pl.BlockSpec((C_MID, 1), full),
          pl.BlockSpec((3, C_MID), full),
          pl.BlockSpec((3, 1), full),
      ],
      out_specs=[row] * 9,
      out_shape=[frow] * 9,
  )(features, pts2, w1, b1_c, g_c, be_c, w2, mol_c)


def _sc_sort_body(keys_hbm, idx_o,
                  keys0, keys1, idx0, idx1, hist,
                  kq, ghist, ck, ci, cbuf,
                  hists_sh, ckeys_sh, cidx_sh, cnts_sh):
  c = lax.axis_index("c")
  s = lax.axis_index("s")
  b = c + 2 * (s % 4)
  bb = s % 4
  q = s // 4

  def zero_hist(nbins):
    def body(i, _):
      hist[pl.ds(i * L, L)] = jnp.zeros((L,), jnp.int32)
      return 0
    lax.fori_loop(0, nbins // L, body, 0)

  def count_sweep(src_k, shift, dmask, n, start=0):
    def body(j, _):
      k = src_k[pl.ds(start + j * L, L)]
      valid = (j * L + lax.iota(jnp.int32, L)) < n
      d = lax.shift_right_logical(k, shift) & dmask
      cnt, lastm = plsc.scan_count(d, mask=valid)
      plsc.addupdate_scatter(hist, [d], cnt, mask=lastm)
      return 0
    lax.fori_loop(0, (n + L - 1) // L, body, 0)

  def excl_scan(nbins):
    def body(i, carry):
      run, m = carry
      chunk = hist[pl.ds(i * L, L)]
      incl = plsc.cumsum(chunk) + run
      hist[pl.ds(i * L, L)] = incl - chunk
      m = jnp.minimum(m, jnp.min(jnp.where(incl >= K, incl, BIG)))
      return jnp.max(incl), m
    _, m = lax.fori_loop(0, nbins // L, body,
                         (jnp.int32(0), jnp.int32(BIG)))
    return m

  def permute_sweep(src_k, src_i, dst_k, dst_i, shift, dmask, n,
                    store_keys=True, start=0):
    def body(j, _):
      k = src_k[pl.ds(start + j * L, L)]
      iv = src_i[pl.ds(start + j * L, L)]
      valid = (j * L + lax.iota(jnp.int32, L)) < n
      d = lax.shift_right_logical(k, shift) & dmask
      cnt, lastm = plsc.scan_count(d, mask=valid)
      base = plsc.load_gather(hist, [d])
      pos = base + cnt - 1
      if store_keys:
        plsc.store_scatter(dst_k, [pos], k, mask=valid)
      plsc.store_scatter(dst_i, [pos], iv, mask=valid)
      plsc.addupdate_scatter(hist, [d], cnt, mask=lastm)
      return 0
    lax.fori_loop(0, (n + L - 1) // L, body, 0)

  QN = N // 4
  QV = QN // L

  pltpu.sync_copy(keys_hbm.at[pl.ds(b * N + q * QN, QN)], kq)
  zero_hist(2048)
  count_sweep(kq, 21, 2047, QN)
  pltpu.sync_copy(hist, hists_sh.at[pl.ds((bb * 4 + q) * 2048, 2048)])
  plsc.subcore_barrier()

  pltpu.sync_copy(hists_sh.at[pl.ds(bb * 4 * 2048, 4 * 2048)], ghist)

  def gscan(i, carry):
    run, db = carry
    tot = (ghist[pl.ds(i * L, L)] + ghist[pl.ds(2048 + i * L, L)] +
           ghist[pl.ds(4096 + i * L, L)] + ghist[pl.ds(6144 + i * L, L)])
    incl = plsc.cumsum(tot) + run
    lane_d = i * L + lax.iota(jnp.int32, L)
    db = jnp.minimum(db, jnp.min(jnp.where(incl >= K, lane_d, BIG)))
    return jnp.max(incl), db
  _, d_bnd = lax.fori_loop(0, 2048 // L, gscan,
                           (jnp.int32(0), jnp.int32(BIG)))

  def compact(j, base):
    k = kq[pl.ds(j * L, L)]
    iv = q * QN + j * L + lax.iota(jnp.int32, L)
    d = lax.shift_right_logical(k, 21) & 2047
    msk = d <= d_bnd
    cnt = plsc.cumsum(jnp.where(msk, 1, 0))
    pos = base + cnt - 1
    plsc.store_scatter(ck, [pos], k, mask=msk)
    plsc.store_scatter(ci, [pos], iv, mask=msk)
    return base + jnp.max(cnt)
  cntq = lax.fori_loop(0, QV, compact, jnp.int32(0))

  pltpu.sync_copy(ck, ckeys_sh.at[pl.ds((bb * 4 + q) * QN, QN)])
  pltpu.sync_copy(ci, cidx_sh.at[pl.ds((bb * 4 + q) * QN, QN)])
  cbuf[pl.ds(0, L)] = jnp.full((L,), cntq, jnp.int32)
  pltpu.sync_copy(cbuf.at[pl.ds(0, L)], cnts_sh.at[pl.ds((bb * 4 + q) * L, L)])
  plsc.subcore_barrier()

  @pl.when(q == 0)
  def _():
    pltpu.sync_copy(ckeys_sh.at[pl.ds(bb * 4 * QN, 4 * QN)], keys1)
    pltpu.sync_copy(cidx_sh.at[pl.ds(bb * 4 * QN, 4 * QN)], idx1)
    pltpu.sync_copy(cnts_sh.at[pl.ds(bb * 4 * L, 4 * L)], cbuf)
    cqs = [jnp.max(cbuf[pl.ds(qq * L, L)]) for qq in range(4)]
    m = cqs[0] + cqs[1] + cqs[2] + cqs[3]

    zero_hist(2048)
    for qq in range(4):
      count_sweep(keys1, 0, 2047, cqs[qq], start=qq * QN)
    excl_scan(2048)
    for qq in range(4):
      permute_sweep(keys1, idx1, keys0, idx0, 0, 2047, cqs[qq],
                    start=qq * QN)

    zero_hist(1024)
    count_sweep(keys0, 11, 1023, m)
    excl_scan(1024)
    permute_sweep(keys0, idx0, keys1, idx1, 11, 1023, m)

    zero_hist(2048)
    count_sweep(keys1, 21, 2047, m)
    excl_scan(2048)
    permute_sweep(keys1, idx1, keys0, idx0, 21, 2047, m, store_keys=False)

    pltpu.sync_copy(idx0.at[pl.ds(0, K)], idx_o.at[pl.ds(b * K, K)])


def _sc_sort_stage(keys):
  mesh = plsc.VectorSubcoreMesh(core_axis_name="c", subcore_axis_name="s",
                                num_cores=SC_CORES, num_subcores=SC_SUBCORES)
  fn = pl.kernel(
      _sc_sort_body,
      out_type=jax.ShapeDtypeStruct((B * K,), jnp.int32),
      mesh=mesh,
      compiler_params=pltpu.CompilerParams(needs_layout_passes=False),
      scratch_types=[
          pltpu.VMEM((N,), jnp.int32),
          pltpu.VMEM((N,), jnp.int32),
          pltpu.VMEM((N,), jnp.int32),
          pltpu.VMEM((N,), jnp.int32),
          pltpu.VMEM((2048,), jnp.int32),
          pltpu.VMEM((N // 4,), jnp.int32),
          pltpu.VMEM((4 * 2048,), jnp.int32),
          pltpu.VMEM((N // 4,), jnp.int32),
          pltpu.VMEM((N // 4,), jnp.int32),
          pltpu.VMEM((64,), jnp.int32),
          pltpu.VMEM_SHARED((4 * 4 * 2048,), jnp.int32),
          pltpu.VMEM_SHARED((4 * N,), jnp.int32),
          pltpu.VMEM_SHARED((4 * N,), jnp.int32),
          pltpu.VMEM_SHARED((4 * 4 * 16,), jnp.int32),
      ],
  )
  return fn(keys)


def _sc_gather_body(idx_hbm, p0, p1, p2, o0, o1, o2, x0, x1, x2,
                    preds_o, orig_o, offs_o, ck, row, outb):
  c = lax.axis_index("c")
  s = lax.axis_index("s")
  b = c + 2 * (s % 4)
  q = s // 4

  pltpu.sync_copy(idx_hbm.at[pl.ds(b * K, K)], ck)
  rows9 = ((p0, preds_o, 0), (p1, preds_o, 1), (p2, preds_o, 2),
           (x0, orig_o, 0), (x1, orig_o, 1), (x2, orig_o, 2),
           (o0, offs_o, 0), (o1, offs_o, 1), (o2, offs_o, 2))
  assign = (0, 1, 2, 3, 0, 1, 2, 3, 0)
  for ridx in range(9):
    src, out_hbm, cc = rows9[ridx]
    @pl.when(q == assign[ridx])
    def _(src=src, out_hbm=out_hbm, cc=cc):
      pltpu.sync_copy(src.at[pl.ds(b * N, N)], row)

      def gather(j, _):
        iv = ck[pl.ds(j * L, L)]
        outb[j // 8, pl.ds((j % 8) * L, L)] = plsc.load_gather(row, [iv])
        return 0
      lax.fori_loop(0, K // L, gather, 0)
      pltpu.sync_copy(outb, out_hbm.at[pl.ds(cc * 32, 32), b])


def _sc_gather_stage(idx, rows9):
  mesh = plsc.VectorSubcoreMesh(core_axis_name="c", subcore_axis_name="s",
                                num_cores=SC_CORES, num_subcores=SC_SUBCORES)
  outk3 = jax.ShapeDtypeStruct((3 * K // 128, B, 128), jnp.float32)
  fn = pl.kernel(
      _sc_gather_body,
      out_type=(outk3, outk3, outk3),
      mesh=mesh,
      compiler_params=pltpu.CompilerParams(needs_layout_passes=False),
      scratch_types=[
          pltpu.VMEM((K,), jnp.int32),
          pltpu.VMEM((N,), jnp.float32),
          pltpu.VMEM((K // 128, 128), jnp.float32),
      ],
  )
  p, o, f = fn(idx, *rows9)
  tok3 = lambda r: jnp.transpose(
      r.reshape(3, K // 128, B, 128), (2, 1, 3, 0)).reshape(B, K, 3)
  return (tok3(p), tok3(o), tok3(f))


def kernel(points, features, cls_preds, W1, b1, gamma, beta, W2,
           max_offset_limit):
  cls2 = jnp.transpose(cls_preds, (2, 0, 1)).reshape(NUM_CLS, B * N)
  pts2 = jnp.transpose(points, (2, 0, 1)).reshape(3, B * N)
  keys = _tc_keys_stage(cls2)
  idx = _sc_sort_stage(keys)
  rows9 = _tc_stage(
      features, pts2, W1,
      b1.reshape(C_MID, 1), gamma.reshape(C_MID, 1), beta.reshape(C_MID, 1),
      W2, max_offset_limit.reshape(3, 1))
  return _sc_gather_stage(idx, rows9)
